# Initial kernel scaffold; baseline (speedup 1.0000x reference)
import jax
import jax.numpy as jnp
from jax.experimental import pallas as pl


def kernel(x, edge_index, W, a):
    raise NotImplementedError("write your pallas kernel here")


# speedup vs baseline: 31.6819x; 31.6819x over previous
import functools

import jax
import jax.numpy as jnp
from jax import lax
from jax.experimental import pallas as pl
from jax.experimental.pallas import tpu as pltpu
from jax.experimental.pallas import tpu_sc as plsc

ALPHA = 0.2

NC = 2
NS = 16
NW = NC * NS
LANES = 16

N_PAD = 10240
ROWS = 80
CHUNK = 128
E_TILE_PAD = ROWS * CHUNK
SLICE = N_PAD // NS


def _mm_body(x_ref, w_ref, a1_ref, a2_ref, wh_ref, s1_ref, s2_ref):
    wh = jnp.dot(x_ref[...], w_ref[...], preferred_element_type=jnp.float32)
    wh_ref[...] = wh
    s1_ref[...] = jnp.dot(wh, a1_ref[...], preferred_element_type=jnp.float32)
    s2_ref[...] = jnp.dot(wh, a2_ref[...], preferred_element_type=jnp.float32)


def _edge_body(s1_hbm, s2_hbm, src_hbm, dst_hbm,
               coef_out, m_out,
               s1_v, s2_v, src_v, dst_v, p_v,
               tmp16_v, allmax_v, slice_v,
               coef_sh, maxes_sh):
    c = lax.axis_index("c")
    s = lax.axis_index("s")
    w = c * NS + s

    pltpu.sync_copy(s1_hbm, s1_v)
    pltpu.sync_copy(s2_hbm, s2_v)
    base = pl.multiple_of(w * ROWS, 8)
    pltpu.sync_copy(src_hbm.at[pl.ds(base, ROWS)], src_v)
    pltpu.sync_copy(dst_hbm.at[pl.ds(base, ROWS)], dst_v)

    z = jnp.zeros((LANES,), jnp.float32)
    for k in range(SLICE // LANES):
        slice_v[pl.ds(k * LANES, LANES)] = z
    pltpu.sync_copy(slice_v, coef_sh.at[pl.ds(s * SLICE, SLICE)])

    def pass1(j, mv):
        for k in range(CHUNK // LANES):
            isrc = src_v[j, pl.ds(k * LANES, LANES)]
            idst = dst_v[j, pl.ds(k * LANES, LANES)]
            ve = plsc.load_gather(s1_v, [isrc]) + plsc.load_gather(s2_v, [idst])
            ve = jnp.maximum(ve, ALPHA * ve)
            p_v[j, pl.ds(k * LANES, LANES)] = ve
            mv = jnp.maximum(mv, ve)
        return mv

    mv = lax.fori_loop(0, ROWS, pass1, jnp.full((LANES,), -3.0e38, jnp.float32))

    tmp16_v[...] = mv
    pltpu.sync_copy(tmp16_v, maxes_sh.at[pl.ds(pl.multiple_of(s * LANES, 8), LANES)])
    plsc.subcore_barrier()
    pltpu.sync_copy(maxes_sh, allmax_v)
    mv = allmax_v[pl.ds(0, LANES)]
    for i in range(1, NS):
        mv = jnp.maximum(mv, allmax_v[pl.ds(i * LANES, LANES)])
    m = jnp.max(mv)
    tmp16_v[...] = mv

    @pl.when(s == 0)
    def _():
        pltpu.sync_copy(tmp16_v, m_out.at[pl.ds(pl.multiple_of(c * LANES, 8), LANES)])

    def pass2(j, carry):
        for k in range(CHUNK // LANES):
            ve = p_v[j, pl.ds(k * LANES, LANES)]
            p_v[j, pl.ds(k * LANES, LANES)] = jnp.exp(ve - m)
        return carry

    lax.fori_loop(0, ROWS, pass2, 0)

    def pass3(j, carry):
        pltpu.sync_copy(p_v.at[j], coef_sh.at[dst_v.at[j]], add=True)
        return carry

    lax.fori_loop(0, ROWS, pass3, 0)
    plsc.subcore_barrier()

    pltpu.sync_copy(coef_sh.at[pl.ds(s * SLICE, SLICE)], slice_v)
    out_off = pl.multiple_of(c * N_PAD + s * SLICE, 8)
    pltpu.sync_copy(slice_v, coef_out.at[pl.ds(out_off, SLICE)])


@functools.cache
def _edge_kernel():
    return functools.partial(
        pl.kernel,
        out_type=[
            jax.ShapeDtypeStruct((NC * N_PAD,), jnp.float32),
            jax.ShapeDtypeStruct((NC * LANES,), jnp.float32),
        ],
        mesh=plsc.VectorSubcoreMesh(
            core_axis_name="c", subcore_axis_name="s", num_cores=NC, num_subcores=NS
        ),
        scratch_types=_edge_scratch(),
        compiler_params=pltpu.CompilerParams(needs_layout_passes=False),
    )(_edge_body)


def _edge_scratch():
    return [
        pltpu.VMEM((N_PAD,), jnp.float32),
        pltpu.VMEM((N_PAD,), jnp.float32),
        pltpu.VMEM((ROWS, CHUNK), jnp.int32),
        pltpu.VMEM((ROWS, CHUNK), jnp.int32),
        pltpu.VMEM((ROWS, CHUNK), jnp.float32),
        pltpu.VMEM((LANES,), jnp.float32),
        pltpu.VMEM((NS * LANES,), jnp.float32),
        pltpu.VMEM((SLICE,), jnp.float32),
        pltpu.VMEM_SHARED((N_PAD,), jnp.float32),
        pltpu.VMEM_SHARED((NS * LANES,), jnp.float32),
    ]


def _fin_body(n_real, wh_ref, coef_ref, m_ref, out_ref):
    mm = m_ref[...]
    big = jnp.max(mm)
    wexp = jnp.exp(jnp.max(mm, axis=0, keepdims=True) - big)
    coef = jnp.sum(coef_ref[...] * wexp, axis=1, keepdims=True)
    idx = lax.broadcasted_iota(jnp.int32, (N_PAD, 1), 0)
    coef = jnp.where(idx < n_real, coef, 0.0)
    denom = jnp.sum(coef)
    out_ref[...] = wh_ref[...] * (coef / denom)


def kernel(x, edge_index, W, a):
    n, f = x.shape
    e = edge_index.shape[1]
    e_tile = e // NW

    src = edge_index[0].astype(jnp.int32)
    dst = edge_index[1].astype(jnp.int32)

    x_pad = jnp.pad(x, ((0, N_PAD - n), (0, 0)))
    a1 = a[0, :f, :]
    a2 = a[0, f:, :]

    wh, s1, s2 = pl.pallas_call(
        _mm_body,
        out_shape=[
            jax.ShapeDtypeStruct((N_PAD, f), jnp.float32),
            jax.ShapeDtypeStruct((N_PAD, 1), jnp.float32),
            jax.ShapeDtypeStruct((N_PAD, 1), jnp.float32),
        ],
    )(x_pad, W[0], a1, a2)

    pad = E_TILE_PAD - e_tile
    srcp = jnp.pad(src.reshape(NW, e_tile), ((0, 0), (0, pad)))
    dump = n + (jnp.arange(pad, dtype=jnp.int32) % (N_PAD - n))
    dstp = jnp.concatenate(
        [dst.reshape(NW, e_tile), jnp.broadcast_to(dump, (NW, pad))], axis=1
    )
    srcp = srcp.reshape(NW * ROWS, CHUNK)
    dstp = dstp.reshape(NW * ROWS, CHUNK)

    coefp, mmax = _edge_kernel()(s1.reshape(N_PAD), s2.reshape(N_PAD), srcp, dstp)

    coef2 = coefp.reshape(NC, N_PAD).T
    m_in = mmax.reshape(NC, LANES).T

    out_full = pl.pallas_call(
        functools.partial(_fin_body, n),
        out_shape=jax.ShapeDtypeStruct((N_PAD, f), jnp.float32),
    )(wh, coef2, m_in)
    return out_full[:n]


# speedup vs baseline: 32.4438x; 1.0241x over previous
import functools

import jax
import jax.numpy as jnp
from jax import lax
from jax.experimental import pallas as pl
from jax.experimental.pallas import tpu as pltpu
from jax.experimental.pallas import tpu_sc as plsc

ALPHA = 0.2
NEG_BIG = -3.0e38

NC = 2
NS = 16
NW = NC * NS
LANES = 16

N_NODES = 10000
N_PAD = 10240
ROWS = 80
CHUNK = 128
E_TILE = 10000
E_TILE_PAD = ROWS * CHUNK
FULL_ROWS = E_TILE // CHUNK
TAIL = E_TILE - FULL_ROWS * CHUNK
SLICE = N_PAD // NS


def _mm_body(x_ref, w_ref, a1_ref, a2_ref, dst_ref,
             wh_ref, s1_ref, s2_ref, dstp_ref):
    wh = jnp.dot(x_ref[...], w_ref[...], preferred_element_type=jnp.float32)
    wh_ref[...] = wh
    s1_ref[...] = jnp.dot(wh, a1_ref[...], preferred_element_type=jnp.float32)
    s2_ref[...] = jnp.dot(wh, a2_ref[...], preferred_element_type=jnp.float32)
    dump = N_NODES + lax.broadcasted_iota(
        jnp.int32, (NW, E_TILE_PAD - E_TILE), 1)
    dstp_ref[...] = jnp.concatenate([dst_ref[...], dump], axis=1)


def _edge_body(s1_hbm, s2_hbm, src_hbm, dstp_hbm,
               coef_out, m_out,
               s1_v, s2_v, src_v, dst_v, p_v,
               tmp16_v, allmax_v, slice_v,
               coef_sh, maxes_sh):
    c = lax.axis_index("c")
    s = lax.axis_index("s")
    w = c * NS + s

    pltpu.sync_copy(s1_hbm, s1_v)
    pltpu.sync_copy(s2_hbm, s2_v)
    pltpu.sync_copy(src_hbm.at[pl.ds(pl.multiple_of(w * E_TILE, 8), E_TILE)],
                    src_v)
    pltpu.sync_copy(dstp_hbm.at[pl.ds(pl.multiple_of(w * ROWS, 8), ROWS)],
                    dst_v)

    z = jnp.zeros((LANES,), jnp.float32)
    for k in range(SLICE // LANES):
        slice_v[pl.ds(k * LANES, LANES)] = z
    pltpu.sync_copy(slice_v, coef_sh.at[pl.ds(s * SLICE, SLICE)])

    def edge_vec(j, k):
        isrc = src_v[pl.ds(j * CHUNK + k * LANES, LANES)]
        idst = dst_v[j, pl.ds(k * LANES, LANES)]
        ve = plsc.load_gather(s1_v, [isrc]) + plsc.load_gather(s2_v, [idst])
        return jnp.maximum(ve, ALPHA * ve)

    def pass1(j, mv):
        for k in range(CHUNK // LANES):
            ve = edge_vec(j, k)
            p_v[j, pl.ds(k * LANES, LANES)] = ve
            mv = jnp.maximum(mv, ve)
        return mv

    mv = lax.fori_loop(0, FULL_ROWS, pass1,
                       jnp.full((LANES,), NEG_BIG, jnp.float32))
    for k in range(TAIL // LANES):
        ve = edge_vec(FULL_ROWS, k)
        p_v[FULL_ROWS, pl.ds(k * LANES, LANES)] = ve
        mv = jnp.maximum(mv, ve)
    pad_e = jnp.full((LANES,), NEG_BIG, jnp.float32)
    for k in range(TAIL // LANES, CHUNK // LANES):
        p_v[FULL_ROWS, pl.ds(k * LANES, LANES)] = pad_e
    for j in range(FULL_ROWS + 1, ROWS):
        for k in range(CHUNK // LANES):
            p_v[j, pl.ds(k * LANES, LANES)] = pad_e

    tmp16_v[...] = mv
    pltpu.sync_copy(
        tmp16_v, maxes_sh.at[pl.ds(pl.multiple_of(s * LANES, 8), LANES)])
    plsc.subcore_barrier()
    pltpu.sync_copy(maxes_sh, allmax_v)
    mv = allmax_v[pl.ds(0, LANES)]
    for i in range(1, NS):
        mv = jnp.maximum(mv, allmax_v[pl.ds(i * LANES, LANES)])
    m = jnp.max(mv)
    tmp16_v[...] = mv

    @pl.when(s == 0)
    def _():
        pltpu.sync_copy(
            tmp16_v, m_out.at[pl.ds(pl.multiple_of(c * LANES, 8), LANES)])

    def pass2(j, carry):
        for k in range(CHUNK // LANES):
            ve = p_v[j, pl.ds(k * LANES, LANES)]
            p_v[j, pl.ds(k * LANES, LANES)] = jnp.exp(ve - m)
        return carry

    lax.fori_loop(0, ROWS, pass2, 0)

    def pass3(j, carry):
        pltpu.sync_copy(p_v.at[j], coef_sh.at[dst_v.at[j]], add=True)
        return carry

    lax.fori_loop(0, ROWS, pass3, 0)
    plsc.subcore_barrier()

    pltpu.sync_copy(coef_sh.at[pl.ds(s * SLICE, SLICE)], slice_v)
    out_off = pl.multiple_of(c * N_PAD + s * SLICE, 8)
    pltpu.sync_copy(slice_v, coef_out.at[pl.ds(out_off, SLICE)])


@functools.cache
def _edge_kernel():
    return functools.partial(
        pl.kernel,
        out_type=[
            jax.ShapeDtypeStruct((NC * N_PAD,), jnp.float32),
            jax.ShapeDtypeStruct((NC * LANES,), jnp.float32),
        ],
        mesh=plsc.VectorSubcoreMesh(
            core_axis_name="c", subcore_axis_name="s", num_cores=NC, num_subcores=NS
        ),
        scratch_types=_edge_scratch(),
        compiler_params=pltpu.CompilerParams(needs_layout_passes=False),
    )(_edge_body)


def _edge_scratch():
    return [
        pltpu.VMEM((N_NODES,), jnp.float32),
        pltpu.VMEM((N_NODES,), jnp.float32),
        pltpu.VMEM((E_TILE,), jnp.int32),
        pltpu.VMEM((ROWS, CHUNK), jnp.int32),
        pltpu.VMEM((ROWS, CHUNK), jnp.float32),
        pltpu.VMEM((LANES,), jnp.float32),
        pltpu.VMEM((NS * LANES,), jnp.float32),
        pltpu.VMEM((SLICE,), jnp.float32),
        pltpu.VMEM_SHARED((N_PAD,), jnp.float32),
        pltpu.VMEM_SHARED((NS * LANES,), jnp.float32),
    ]


def _fin_body(wh_ref, coef_ref, m_ref, out_ref):
    mm = m_ref[...]
    big = jnp.max(mm)
    wexp = jnp.exp(jnp.max(mm, axis=0, keepdims=True) - big)
    coef = jnp.sum(coef_ref[...] * wexp, axis=1, keepdims=True)
    denom = jnp.sum(coef)
    out_ref[...] = wh_ref[...] * (coef / denom)


def kernel(x, edge_index, W, a):
    n, f = x.shape
    src = edge_index[0].astype(jnp.int32)
    dst2d = edge_index[1].astype(jnp.int32).reshape(NW, E_TILE)

    a1 = a[0, :f, :]
    a2 = a[0, f:, :]

    wh, s1, s2, dstp = pl.pallas_call(
        _mm_body,
        out_shape=[
            jax.ShapeDtypeStruct((n, f), jnp.float32),
            jax.ShapeDtypeStruct((n, 1), jnp.float32),
            jax.ShapeDtypeStruct((n, 1), jnp.float32),
            jax.ShapeDtypeStruct((NW, E_TILE_PAD), jnp.int32),
        ],
    )(x, W[0], a1, a2, dst2d)

    coefp, mmax = _edge_kernel()(
        s1.reshape(n), s2.reshape(n), src, dstp.reshape(NW * ROWS, CHUNK))

    coef2 = coefp.reshape(NC, N_PAD)[:, :n].T
    m_in = mmax.reshape(NC, LANES).T

    return pl.pallas_call(
        _fin_body,
        out_shape=jax.ShapeDtypeStruct((n, f), jnp.float32),
    )(wh, coef2, m_in)


# speedup vs baseline: 35.9348x; 1.1076x over previous
import functools

import jax
import jax.numpy as jnp
import numpy as np
from jax import lax
from jax.experimental import pallas as pl
from jax.experimental.pallas import tpu as pltpu
from jax.experimental.pallas import tpu_sc as plsc

ALPHA = 0.2
NEG_BIG = -3.0e38

NC = 2
NS = 16
NW = NC * NS
LANES = 16

N_NODES = 10000
N_PAD = 10240
ROWS = 80
CHUNK = 128
E_TILE_PAD = ROWS * CHUNK
E_PAD_TOTAL = NW * E_TILE_PAD
TBL = 2 * N_PAD
TBL_CHUNK = TBL // NS
SLICE = N_PAD // NS
GRID = 10
BLK = N_NODES // GRID

_PAD_E = E_PAD_TOTAL - 320000
_PAD_BLOCK = np.stack([
    np.zeros((_PAD_E,), np.int32),
    (N_NODES + np.arange(_PAD_E) % (N_PAD - N_NODES)).astype(np.int32),
])


def _mm_body(x_ref, w_ref, a12_ref, wh_ref, s12_ref):
    wh = jnp.dot(x_ref[...], w_ref[...], preferred_element_type=jnp.float32)
    wh_ref[...] = wh
    s12_ref[...] = jnp.dot(wh, a12_ref[...], preferred_element_type=jnp.float32)


def _edge_body(t_hbm, ei_hbm,
               coef_out, m_out,
               t_v, src_v, dst_v, p_v,
               tmp16_v, allmax_v, slice_v,
               coef_sh, maxes_sh,
               sem):
    c = lax.axis_index("c")
    s = lax.axis_index("s")
    w = c * NS + s
    base = pl.multiple_of(w * ROWS, 8)

    for k in range(NS):
        r = s + k
        r = jnp.where(r >= NS, r - NS, r)
        off = pl.multiple_of(r * TBL_CHUNK, 8)
        pltpu.async_copy(t_hbm.at[pl.ds(off, TBL_CHUNK)],
                         t_v.at[pl.ds(off, TBL_CHUNK)], sem)
    pltpu.async_copy(ei_hbm.at[0, pl.ds(base, ROWS)], src_v, sem)
    pltpu.async_copy(ei_hbm.at[1, pl.ds(base, ROWS)], dst_v, sem)

    z = jnp.zeros((LANES,), jnp.float32)
    for k in range(SLICE // LANES):
        slice_v[pl.ds(k * LANES, LANES)] = z

    pltpu.make_async_copy(t_hbm, t_v, sem).wait()
    pltpu.make_async_copy(ei_hbm.at[0, pl.ds(base, ROWS)], src_v, sem).wait()
    pltpu.make_async_copy(ei_hbm.at[1, pl.ds(base, ROWS)], dst_v, sem).wait()

    pltpu.sync_copy(slice_v, coef_sh.at[pl.ds(s * SLICE, SLICE)])

    def pass1(j, mv):
        for k in range(CHUNK // LANES):
            isrc = src_v[j, pl.ds(k * LANES, LANES)]
            idst = dst_v[j, pl.ds(k * LANES, LANES)]
            ve = (plsc.load_gather(t_v, [isrc * 2])
                  + plsc.load_gather(t_v, [idst * 2 + 1]))
            ve = jnp.maximum(ve, ALPHA * ve)
            p_v[j, pl.ds(k * LANES, LANES)] = ve
            mv = jnp.maximum(mv, ve)
        return mv

    mv = lax.fori_loop(0, ROWS, pass1,
                       jnp.full((LANES,), NEG_BIG, jnp.float32))

    tmp16_v[...] = mv
    pltpu.sync_copy(
        tmp16_v, maxes_sh.at[pl.ds(pl.multiple_of(s * LANES, 8), LANES)])
    plsc.subcore_barrier()
    pltpu.sync_copy(maxes_sh, allmax_v)
    mv = allmax_v[pl.ds(0, LANES)]
    for i in range(1, NS):
        mv = jnp.maximum(mv, allmax_v[pl.ds(i * LANES, LANES)])
    m = jnp.max(mv)
    tmp16_v[...] = mv

    @pl.when(s == 0)
    def _():
        pltpu.sync_copy(
            tmp16_v, m_out.at[pl.ds(pl.multiple_of(c * LANES, 8), LANES)])

    def pass2(j, carry):
        for k in range(CHUNK // LANES):
            ve = p_v[j, pl.ds(k * LANES, LANES)]
            p_v[j, pl.ds(k * LANES, LANES)] = jnp.exp(ve - m)
        pltpu.async_copy(p_v.at[j], coef_sh.at[dst_v.at[j]], sem, add=True)
        return carry

    lax.fori_loop(0, ROWS, pass2, 0)
    pltpu.make_async_copy(t_hbm.at[pl.ds(0, E_TILE_PAD)],
                          t_v.at[pl.ds(0, E_TILE_PAD)], sem).wait()
    plsc.subcore_barrier()

    pltpu.sync_copy(coef_sh.at[pl.ds(s * SLICE, SLICE)], slice_v)
    out_off = pl.multiple_of(c * N_PAD + s * SLICE, 8)
    pltpu.sync_copy(slice_v, coef_out.at[pl.ds(out_off, SLICE)])


@functools.cache
def _edge_kernel():
    return functools.partial(
        pl.kernel,
        out_type=[
            jax.ShapeDtypeStruct((NC * N_PAD,), jnp.float32),
            jax.ShapeDtypeStruct((NC * LANES,), jnp.float32),
        ],
        mesh=plsc.VectorSubcoreMesh(
            core_axis_name="c", subcore_axis_name="s", num_cores=NC, num_subcores=NS
        ),
        scratch_types=_edge_scratch(),
        compiler_params=pltpu.CompilerParams(needs_layout_passes=False),
    )(_edge_body)


def _edge_scratch():
    return [
        pltpu.VMEM((TBL,), jnp.float32),
        pltpu.VMEM((ROWS, CHUNK), jnp.int32),
        pltpu.VMEM((ROWS, CHUNK), jnp.int32),
        pltpu.VMEM((ROWS, CHUNK), jnp.float32),
        pltpu.VMEM((LANES,), jnp.float32),
        pltpu.VMEM((NS * LANES,), jnp.float32),
        pltpu.VMEM((SLICE,), jnp.float32),
        pltpu.VMEM_SHARED((N_PAD,), jnp.float32),
        pltpu.VMEM_SHARED((NS * LANES,), jnp.float32),
        pltpu.SemaphoreType.DMA,
    ]


def _fin_body(wh_ref, coef_ref, m_ref, out_ref):
    i = pl.program_id(0)
    mm = m_ref[...]
    big = jnp.max(mm)
    wexp = jnp.exp(jnp.max(mm, axis=0, keepdims=True) - big)
    cf = coef_ref[...] * wexp
    denom = jnp.sum(cf)
    cb = coef_ref[pl.ds(i * BLK, BLK), :] * wexp
    coef = jnp.sum(cb, axis=1, keepdims=True)
    out_ref[...] = wh_ref[...] * (coef / denom)


def kernel(x, edge_index, W, a):
    n, f = x.shape
    a12 = jnp.concatenate([a[0, :f, :], a[0, f:, :]], axis=1)

    wh, s12 = pl.pallas_call(
        _mm_body,
        grid=(GRID,),
        in_specs=[
            pl.BlockSpec((BLK, f), lambda i: (i, 0)),
            pl.BlockSpec((f, f), lambda i: (0, 0)),
            pl.BlockSpec((f, 2), lambda i: (0, 0)),
        ],
        out_specs=[
            pl.BlockSpec((BLK, f), lambda i: (i, 0)),
            pl.BlockSpec((BLK, 2), lambda i: (i, 0)),
        ],
        out_shape=[
            jax.ShapeDtypeStruct((n, f), jnp.float32),
            jax.ShapeDtypeStruct((n, 2), jnp.float32),
        ],
    )(x, W[0], a12)

    t = jnp.pad(s12, ((0, N_PAD - n), (0, 0))).reshape(TBL)
    ei3 = jnp.concatenate(
        [edge_index.astype(jnp.int32), jnp.asarray(_PAD_BLOCK)], axis=1
    ).reshape(2, NW * ROWS, CHUNK)

    coefp, mmax = _edge_kernel()(t, ei3)

    coef2 = coefp.reshape(NC, N_PAD)[:, :n].T
    m_in = mmax.reshape(NC, LANES).T

    return pl.pallas_call(
        _fin_body,
        grid=(GRID,),
        in_specs=[
            pl.BlockSpec((BLK, f), lambda i: (i, 0)),
            pl.BlockSpec((n, NC), lambda i: (0, 0)),
            pl.BlockSpec((LANES, NC), lambda i: (0, 0)),
        ],
        out_specs=pl.BlockSpec((BLK, f), lambda i: (i, 0)),
        out_shape=jax.ShapeDtypeStruct((n, f), jnp.float32),
    )(wh, coef2, m_in)


# speedup vs baseline: 49.6540x; 1.3818x over previous
import functools

import jax
import jax.numpy as jnp
import numpy as np
from jax import lax
from jax.experimental import pallas as pl
from jax.experimental.pallas import tpu as pltpu
from jax.experimental.pallas import tpu_sc as plsc

ALPHA = 0.2
NEG_BIG = -3.0e38

NC = 2
NS = 16
NW = NC * NS
LANES = 16

N_NODES = 10000
N_PAD = 10240
ROWS = 80
CHUNK = 128
E_TILE_PAD = ROWS * CHUNK
E_PAD_TOTAL = NW * E_TILE_PAD
TBL_CHUNK = N_PAD // NS
SLICE = N_PAD // NS
GRID = 10
BLK = N_PAD // GRID

_PAD_E = E_PAD_TOTAL - 320000
_PAD_BLOCK = np.stack([
    np.zeros((_PAD_E,), np.int32),
    (N_NODES + np.arange(_PAD_E) % (N_PAD - N_NODES)).astype(np.int32),
])


def _mm_body(x_ref, w_ref, a12_ref, wh_ref, s1_ref, s2_ref):
    wh = jnp.dot(x_ref[...], w_ref[...], preferred_element_type=jnp.float32)
    wh_ref[...] = wh
    srow = lax.dot_general(a12_ref[...], wh, (((0,), (1,)), ((), ())),
                           preferred_element_type=jnp.float32)
    z7 = jnp.zeros((7, BLK), jnp.float32)
    s1_ref[...] = jnp.concatenate([srow[0:1, :], z7], axis=0)
    s2_ref[...] = jnp.concatenate([srow[1:2, :], z7], axis=0)


def _edge_body(s1_hbm, s2_hbm, ei_hbm,
               coef_out, m_out,
               s1_v, s2_v, src_v, dst_v, p_v,
               tmp16_v, allmax_v, slice_v,
               coef_sh, maxes_sh,
               sem):
    c = lax.axis_index("c")
    s = lax.axis_index("s")
    w = c * NS + s
    base = pl.multiple_of(w * ROWS, 8)

    for k in range(NS):
        r = s + k
        r = jnp.where(r >= NS, r - NS, r)
        off = pl.multiple_of(r * TBL_CHUNK, 8)
        pltpu.async_copy(s1_hbm.at[0, pl.ds(off, TBL_CHUNK)],
                         s1_v.at[pl.ds(off, TBL_CHUNK)], sem)
        pltpu.async_copy(s2_hbm.at[0, pl.ds(off, TBL_CHUNK)],
                         s2_v.at[pl.ds(off, TBL_CHUNK)], sem)
    pltpu.async_copy(ei_hbm.at[0, pl.ds(base, ROWS)], src_v, sem)
    pltpu.async_copy(ei_hbm.at[1, pl.ds(base, ROWS)], dst_v, sem)

    z = jnp.zeros((LANES,), jnp.float32)
    for k in range(SLICE // LANES):
        slice_v[pl.ds(k * LANES, LANES)] = z

    pltpu.make_async_copy(s1_hbm.at[0], s1_v, sem).wait()
    pltpu.make_async_copy(s2_hbm.at[0], s2_v, sem).wait()
    pltpu.make_async_copy(ei_hbm.at[0, pl.ds(base, ROWS)], src_v, sem).wait()
    pltpu.make_async_copy(ei_hbm.at[1, pl.ds(base, ROWS)], dst_v, sem).wait()

    pltpu.sync_copy(slice_v, coef_sh.at[pl.ds(s * SLICE, SLICE)])

    def pass1(j, mv):
        for k in range(CHUNK // LANES):
            isrc = src_v[j, pl.ds(k * LANES, LANES)]
            idst = dst_v[j, pl.ds(k * LANES, LANES)]
            ve = (plsc.load_gather(s1_v, [isrc])
                  + plsc.load_gather(s2_v, [idst]))
            ve = jnp.maximum(ve, ALPHA * ve)
            p_v[j, pl.ds(k * LANES, LANES)] = ve
            mv = jnp.maximum(mv, ve)
        return mv

    mv = lax.fori_loop(0, ROWS, pass1,
                       jnp.full((LANES,), NEG_BIG, jnp.float32))

    tmp16_v[...] = mv
    pltpu.sync_copy(
        tmp16_v, maxes_sh.at[pl.ds(pl.multiple_of(s * LANES, 8), LANES)])
    plsc.subcore_barrier()
    pltpu.sync_copy(maxes_sh, allmax_v)
    mv = allmax_v[pl.ds(0, LANES)]
    for i in range(1, NS):
        mv = jnp.maximum(mv, allmax_v[pl.ds(i * LANES, LANES)])
    m = jnp.max(mv)
    tmp16_v[...] = mv

    @pl.when(s == 0)
    def _():
        pltpu.sync_copy(
            tmp16_v, m_out.at[pl.multiple_of(8 * c, 8), pl.ds(0, LANES)])

    def pass2(j, carry):
        for k in range(CHUNK // LANES):
            ve = p_v[j, pl.ds(k * LANES, LANES)]
            p_v[j, pl.ds(k * LANES, LANES)] = jnp.exp(ve - m)
        pltpu.async_copy(p_v.at[j], coef_sh.at[dst_v.at[j]], sem, add=True)
        return carry

    lax.fori_loop(0, ROWS, pass2, 0)
    pltpu.make_async_copy(s1_hbm.at[0], s1_v, sem).wait()
    plsc.subcore_barrier()

    pltpu.sync_copy(coef_sh.at[pl.ds(s * SLICE, SLICE)], slice_v)
    pltpu.sync_copy(
        slice_v,
        coef_out.at[pl.multiple_of(8 * c, 8), pl.ds(s * SLICE, SLICE)])


@functools.cache
def _edge_kernel():
    return functools.partial(
        pl.kernel,
        out_type=[
            jax.ShapeDtypeStruct((2 * 8, N_PAD), jnp.float32),
            jax.ShapeDtypeStruct((2 * 8, LANES), jnp.float32),
        ],
        mesh=plsc.VectorSubcoreMesh(
            core_axis_name="c", subcore_axis_name="s", num_cores=NC, num_subcores=NS
        ),
        scratch_types=_edge_scratch(),
        compiler_params=pltpu.CompilerParams(needs_layout_passes=False),
    )(_edge_body)


def _edge_scratch():
    return [
        pltpu.VMEM((N_PAD,), jnp.float32),
        pltpu.VMEM((N_PAD,), jnp.float32),
        pltpu.VMEM((ROWS, CHUNK), jnp.int32),
        pltpu.VMEM((ROWS, CHUNK), jnp.int32),
        pltpu.VMEM((ROWS, CHUNK), jnp.float32),
        pltpu.VMEM((LANES,), jnp.float32),
        pltpu.VMEM((NS * LANES,), jnp.float32),
        pltpu.VMEM((SLICE,), jnp.float32),
        pltpu.VMEM_SHARED((N_PAD,), jnp.float32),
        pltpu.VMEM_SHARED((NS * LANES,), jnp.float32),
        pltpu.SemaphoreType.DMA,
    ]


def _fin_body(wh_ref, coef_ref, m_ref, out_ref):
    m0 = jnp.max(m_ref[0:1, :])
    m1 = jnp.max(m_ref[8:9, :])
    big = jnp.maximum(m0, m1)
    w0 = jnp.exp(m0 - big)
    w1 = jnp.exp(m1 - big)
    row = w0 * coef_ref[0:1, :] + w1 * coef_ref[8:9, :]
    denom = jnp.sum(row[:, :N_NODES])
    col = jnp.transpose(row, (1, 0))
    out_ref[...] = wh_ref[0:N_NODES, :] * (col[0:N_NODES, :] / denom)


def kernel(x, edge_index, W, a):
    n, f = x.shape
    a12 = jnp.concatenate([a[0, :f, :], a[0, f:, :]], axis=1)
    x_pad = jnp.pad(x, ((0, N_PAD - n), (0, 0)))

    wh, s1t, s2t = pl.pallas_call(
        _mm_body,
        grid=(GRID,),
        in_specs=[
            pl.BlockSpec((BLK, f), lambda i: (i, 0)),
            pl.BlockSpec((f, f), lambda i: (0, 0)),
            pl.BlockSpec((f, 2), lambda i: (0, 0)),
        ],
        out_specs=[
            pl.BlockSpec((BLK, f), lambda i: (i, 0)),
            pl.BlockSpec((8, BLK), lambda i: (0, i)),
            pl.BlockSpec((8, BLK), lambda i: (0, i)),
        ],
        out_shape=[
            jax.ShapeDtypeStruct((N_PAD, f), jnp.float32),
            jax.ShapeDtypeStruct((8, N_PAD), jnp.float32),
            jax.ShapeDtypeStruct((8, N_PAD), jnp.float32),
        ],
    )(x_pad, W[0], a12)

    ei3 = jnp.concatenate(
        [edge_index.astype(jnp.int32), jnp.asarray(_PAD_BLOCK)], axis=1
    ).reshape(2, NW * ROWS, CHUNK)

    coef_t, m_t = _edge_kernel()(s1t, s2t, ei3)

    return pl.pallas_call(
        _fin_body,
        out_shape=jax.ShapeDtypeStruct((n, f), jnp.float32),
    )(wh, coef_t, m_t)


# speedup vs baseline: 52.1547x; 1.0504x over previous
import functools

import jax
import jax.numpy as jnp
import numpy as np
from jax import lax
from jax.experimental import pallas as pl
from jax.experimental.pallas import tpu as pltpu
from jax.experimental.pallas import tpu_sc as plsc

ALPHA = 0.2
NEG_BIG = -3.0e38

NC = 2
NS = 16
NW = NC * NS
LANES = 16

N_NODES = 10000
N_PAD = 10240
ROWS = 80
CHUNK = 128
E_TILE_PAD = ROWS * CHUNK
E_PAD_TOTAL = NW * E_TILE_PAD
TBL_CHUNK = N_PAD // NS
SLICE = N_PAD // NS
GRID = 10
BLK = N_PAD // GRID

_PAD_E = E_PAD_TOTAL - 320000
_PAD_BLOCK = np.stack([
    np.zeros((_PAD_E,), np.int32),
    (N_NODES + np.arange(_PAD_E) % (N_PAD - N_NODES)).astype(np.int32),
])


def _mm_body(x_ref, w_ref, a_ref, s1_ref, s2_ref, m_ref):
    i = pl.program_id(0)
    wh = jnp.dot(x_ref[...], w_ref[...], preferred_element_type=jnp.float32)
    a12 = jnp.concatenate([a_ref[0:128, :], a_ref[128:256, :]], axis=1)
    srow = lax.dot_general(a12, wh, (((0,), (1,)), ((), ())),
                           preferred_element_type=jnp.float32)
    g = i * BLK + lax.broadcasted_iota(jnp.int32, (1, BLK), 1)
    valid = g < N_NODES
    srow_z = jnp.where(valid, srow, 0.0)
    z7 = jnp.zeros((7, BLK), jnp.float32)
    s1_ref[...] = jnp.concatenate([srow_z[0:1, :], z7], axis=0)
    s2_ref[...] = jnp.concatenate([srow_z[1:2, :], z7], axis=0)

    srow_m = jnp.where(valid, srow, NEG_BIG)
    ms1 = jnp.max(srow_m[0:1, :])
    ms2 = jnp.max(srow_m[1:2, :])
    mblk = jnp.concatenate(
        [jnp.full((4, 128), ms1), jnp.full((4, 128), ms2)], axis=0)

    @pl.when(i == 0)
    def _():
        m_ref[...] = jnp.full((8, 128), NEG_BIG)

    m_ref[...] = jnp.maximum(m_ref[...], mblk)


def _edge_body(s1_hbm, s2_hbm, m_hbm, ei_hbm,
               coef_out,
               s1_v, s2_v, m_v, src_v, dst_v, p_v,
               slice_v,
               coef_sh,
               sem):
    c = lax.axis_index("c")
    s = lax.axis_index("s")
    w = c * NS + s
    base = pl.multiple_of(w * ROWS, 8)

    for k in range(NS):
        r = s + k
        r = jnp.where(r >= NS, r - NS, r)
        off = pl.multiple_of(r * TBL_CHUNK, 8)
        pltpu.async_copy(s1_hbm.at[0, pl.ds(off, TBL_CHUNK)],
                         s1_v.at[pl.ds(off, TBL_CHUNK)], sem)
        pltpu.async_copy(s2_hbm.at[0, pl.ds(off, TBL_CHUNK)],
                         s2_v.at[pl.ds(off, TBL_CHUNK)], sem)
    pltpu.async_copy(ei_hbm.at[0, pl.ds(base, ROWS)], src_v, sem)
    pltpu.async_copy(ei_hbm.at[1, pl.ds(base, ROWS)], dst_v, sem)
    pltpu.sync_copy(m_hbm, m_v)

    z = jnp.zeros((LANES,), jnp.float32)
    for k in range(SLICE // LANES):
        slice_v[pl.ds(k * LANES, LANES)] = z

    ms1 = jnp.max(m_v[0, pl.ds(0, LANES)])
    ms2 = jnp.max(m_v[4, pl.ds(0, LANES)])
    m = ms1 + ms2
    m = jnp.maximum(m, ALPHA * m)

    pltpu.make_async_copy(s1_hbm.at[0], s1_v, sem).wait()
    pltpu.make_async_copy(s2_hbm.at[0], s2_v, sem).wait()
    pltpu.make_async_copy(ei_hbm.at[0, pl.ds(base, ROWS)], src_v, sem).wait()
    pltpu.make_async_copy(ei_hbm.at[1, pl.ds(base, ROWS)], dst_v, sem).wait()

    pltpu.sync_copy(slice_v, coef_sh.at[pl.ds(s * SLICE, SLICE)])
    plsc.subcore_barrier()

    def edges(j, carry):
        for k in range(CHUNK // LANES):
            isrc = src_v[j, pl.ds(k * LANES, LANES)]
            idst = dst_v[j, pl.ds(k * LANES, LANES)]
            ve = (plsc.load_gather(s1_v, [isrc])
                  + plsc.load_gather(s2_v, [idst]))
            ve = jnp.maximum(ve, ALPHA * ve)
            p_v[j, pl.ds(k * LANES, LANES)] = jnp.exp(ve - m)
        pltpu.async_copy(p_v.at[j], coef_sh.at[dst_v.at[j]], sem, add=True)
        return carry

    lax.fori_loop(0, ROWS, edges, 0)
    pltpu.make_async_copy(s1_hbm.at[0], s1_v, sem).wait()
    plsc.subcore_barrier()

    pltpu.sync_copy(coef_sh.at[pl.ds(s * SLICE, SLICE)], slice_v)
    pltpu.sync_copy(
        slice_v,
        coef_out.at[pl.multiple_of(8 * c, 8), pl.ds(s * SLICE, SLICE)])


@functools.cache
def _edge_kernel():
    return functools.partial(
        pl.kernel,
        out_type=jax.ShapeDtypeStruct((2 * 8, N_PAD), jnp.float32),
        mesh=plsc.VectorSubcoreMesh(
            core_axis_name="c", subcore_axis_name="s", num_cores=NC, num_subcores=NS
        ),
        scratch_types=_edge_scratch(),
        compiler_params=pltpu.CompilerParams(needs_layout_passes=False),
    )(_edge_body)


def _edge_scratch():
    return [
        pltpu.VMEM((N_PAD,), jnp.float32),
        pltpu.VMEM((N_PAD,), jnp.float32),
        pltpu.VMEM((8, 128), jnp.float32),
        pltpu.VMEM((ROWS, CHUNK), jnp.int32),
        pltpu.VMEM((ROWS, CHUNK), jnp.int32),
        pltpu.VMEM((ROWS, CHUNK), jnp.float32),
        pltpu.VMEM((SLICE,), jnp.float32),
        pltpu.VMEM_SHARED((N_PAD,), jnp.float32),
        pltpu.SemaphoreType.DMA,
    ]


def _fin_body(x_ref, w_ref, coef_ref, out_ref):
    wh = jnp.dot(x_ref[...], w_ref[...], preferred_element_type=jnp.float32)
    row = coef_ref[0:1, :] + coef_ref[8:9, :]
    denom = jnp.sum(row[:, :N_NODES])
    col = jnp.transpose(row, (1, 0))
    out_ref[...] = wh * (col[0:N_NODES, :] / denom)


def kernel(x, edge_index, W, a):
    n, f = x.shape

    s1t, s2t, m_t = pl.pallas_call(
        _mm_body,
        grid=(GRID,),
        in_specs=[
            pl.BlockSpec((BLK, f), lambda i: (i, 0)),
            pl.BlockSpec((f, f), lambda i: (0, 0)),
            pl.BlockSpec((2 * f, 1), lambda i: (0, 0)),
        ],
        out_specs=[
            pl.BlockSpec((8, BLK), lambda i: (0, i)),
            pl.BlockSpec((8, BLK), lambda i: (0, i)),
            pl.BlockSpec((8, 128), lambda i: (0, 0)),
        ],
        out_shape=[
            jax.ShapeDtypeStruct((8, N_PAD), jnp.float32),
            jax.ShapeDtypeStruct((8, N_PAD), jnp.float32),
            jax.ShapeDtypeStruct((8, 128), jnp.float32),
        ],
    )(x, W[0], a[0])

    ei3 = jnp.concatenate(
        [edge_index.astype(jnp.int32), jnp.asarray(_PAD_BLOCK)], axis=1
    ).reshape(2, NW * ROWS, CHUNK)

    coef_t = _edge_kernel()(s1t, s2t, m_t, ei3)

    return pl.pallas_call(
        _fin_body,
        out_shape=jax.ShapeDtypeStruct((n, f), jnp.float32),
    )(x, W[0], coef_t)


# speedup vs baseline: 53.6009x; 1.0277x over previous
import functools

import jax
import jax.numpy as jnp
import numpy as np
from jax import lax
from jax.experimental import pallas as pl
from jax.experimental.pallas import tpu as pltpu
from jax.experimental.pallas import tpu_sc as plsc

ALPHA = 0.2
NEG_BIG = -3.0e38

NC = 2
NS = 16
NW = NC * NS
LANES = 16

N_NODES = 10000
N_PAD = 10240
ROWS = 80
CHUNK = 128
E_TILE_PAD = ROWS * CHUNK
E_PAD_TOTAL = NW * E_TILE_PAD
TBL_CHUNK = N_PAD // NS
SLICE = N_PAD // NS
GRID = 10
BLK = N_PAD // GRID

E_ROWS = 320000 // CHUNK
FULL_TILES = E_ROWS // ROWS
TAIL_ROWS = E_ROWS - FULL_TILES * ROWS


def _mm_body(x_ref, w_ref, a_ref, s1_ref, s2_ref, m_ref):
    i = pl.program_id(0)
    wh = jnp.dot(x_ref[...], w_ref[...], preferred_element_type=jnp.float32)
    srow = lax.dot_general(a_ref[...], wh, (((1,), (1,)), ((), ())),
                           preferred_element_type=jnp.float32)
    g = i * BLK + lax.broadcasted_iota(jnp.int32, (1, BLK), 1)
    valid = g < N_NODES
    srow_z = jnp.where(valid, srow, 0.0)
    z7 = jnp.zeros((7, BLK), jnp.float32)
    s1_ref[...] = jnp.concatenate([srow_z[0:1, :], z7], axis=0)
    s2_ref[...] = jnp.concatenate([srow_z[1:2, :], z7], axis=0)

    srow_m = jnp.where(valid, srow, NEG_BIG)
    ms1 = jnp.max(srow_m[0:1, :])
    ms2 = jnp.max(srow_m[1:2, :])
    mblk = jnp.concatenate(
        [jnp.full((4, 128), ms1), jnp.full((4, 128), ms2)], axis=0)

    @pl.when(i == 0)
    def _():
        m_ref[...] = jnp.full((8, 128), NEG_BIG)

    m_ref[...] = jnp.maximum(m_ref[...], mblk)


def _edge_body(s1_hbm, s2_hbm, m_hbm, ej_hbm,
               coef_out,
               s1_v, s2_v, m_v, ej_v, p_v,
               slice_v,
               coef_sh,
               sem):
    c = lax.axis_index("c")
    s = lax.axis_index("s")
    w = c * NS + s
    base = pl.multiple_of(w * ROWS, 8)

    for k in range(NS):
        r = s + k
        r = jnp.where(r >= NS, r - NS, r)
        off = pl.multiple_of(r * TBL_CHUNK, 8)
        pltpu.async_copy(s1_hbm.at[0, pl.ds(off, TBL_CHUNK)],
                         s1_v.at[pl.ds(off, TBL_CHUNK)], sem)
        pltpu.async_copy(s2_hbm.at[0, pl.ds(off, TBL_CHUNK)],
                         s2_v.at[pl.ds(off, TBL_CHUNK)], sem)

    @pl.when(w < FULL_TILES)
    def _():
        pltpu.async_copy(ej_hbm.at[pl.ds(base, ROWS)], ej_v, sem)

    @pl.when(w == FULL_TILES)
    def _():
        pltpu.async_copy(ej_hbm.at[pl.ds(base, TAIL_ROWS)],
                         ej_v.at[pl.ds(0, TAIL_ROWS)], sem)

    pltpu.sync_copy(m_hbm, m_v)

    z = jnp.zeros((LANES,), jnp.float32)
    for k in range(SLICE // LANES):
        slice_v[pl.ds(k * LANES, LANES)] = z

    @pl.when(w == FULL_TILES)
    def _():
        zi = jnp.zeros((LANES,), jnp.int32)
        lane = lax.iota(jnp.int32, LANES)

        def fill(j, carry):
            for k in range(CHUNK // LANES):
                dump = N_NODES + lax.rem(j * CHUNK + k * LANES + lane,
                                         N_PAD - N_NODES)
                ej_v[j, 0, pl.ds(k * LANES, LANES)] = zi
                ej_v[j, 1, pl.ds(k * LANES, LANES)] = dump
            return carry

        lax.fori_loop(TAIL_ROWS, ROWS, fill, 0)

    ms1 = jnp.max(m_v[0, pl.ds(0, LANES)])
    ms2 = jnp.max(m_v[4, pl.ds(0, LANES)])
    m = ms1 + ms2
    m = jnp.maximum(m, ALPHA * m)

    pltpu.make_async_copy(s1_hbm.at[0], s1_v, sem).wait()
    pltpu.make_async_copy(s2_hbm.at[0], s2_v, sem).wait()

    @pl.when(w < FULL_TILES)
    def _():
        pltpu.make_async_copy(ej_hbm.at[pl.ds(base, ROWS)], ej_v, sem).wait()

    @pl.when(w == FULL_TILES)
    def _():
        pltpu.make_async_copy(ej_hbm.at[pl.ds(base, TAIL_ROWS)],
                              ej_v.at[pl.ds(0, TAIL_ROWS)], sem).wait()

    pltpu.sync_copy(slice_v, coef_sh.at[pl.ds(s * SLICE, SLICE)])
    plsc.subcore_barrier()

    def edges(j, carry):
        for k in range(CHUNK // LANES):
            isrc = ej_v[j, 0, pl.ds(k * LANES, LANES)]
            idst = ej_v[j, 1, pl.ds(k * LANES, LANES)]
            ve = (plsc.load_gather(s1_v, [isrc])
                  + plsc.load_gather(s2_v, [idst]))
            ve = jnp.maximum(ve, ALPHA * ve)
            p_v[j, pl.ds(k * LANES, LANES)] = jnp.exp(ve - m)
        pltpu.async_copy(p_v.at[j], coef_sh.at[ej_v.at[j, 1]], sem, add=True)
        return carry

    lax.fori_loop(0, ROWS, edges, 0)
    pltpu.make_async_copy(s1_hbm.at[0], s1_v, sem).wait()
    plsc.subcore_barrier()

    pltpu.sync_copy(coef_sh.at[pl.ds(s * SLICE, SLICE)], slice_v)
    pltpu.sync_copy(
        slice_v,
        coef_out.at[pl.multiple_of(8 * c, 8), pl.ds(s * SLICE, SLICE)])


@functools.cache
def _edge_kernel():
    return functools.partial(
        pl.kernel,
        out_type=jax.ShapeDtypeStruct((2 * 8, N_PAD), jnp.float32),
        mesh=plsc.VectorSubcoreMesh(
            core_axis_name="c", subcore_axis_name="s", num_cores=NC, num_subcores=NS
        ),
        scratch_types=_edge_scratch(),
        compiler_params=pltpu.CompilerParams(needs_layout_passes=False),
    )(_edge_body)


def _edge_scratch():
    return [
        pltpu.VMEM((N_PAD,), jnp.float32),
        pltpu.VMEM((N_PAD,), jnp.float32),
        pltpu.VMEM((8, 128), jnp.float32),
        pltpu.VMEM((ROWS, 2, CHUNK), jnp.int32),
        pltpu.VMEM((ROWS, CHUNK), jnp.float32),
        pltpu.VMEM((SLICE,), jnp.float32),
        pltpu.VMEM_SHARED((N_PAD,), jnp.float32),
        pltpu.SemaphoreType.DMA,
    ]


BLKC = N_NODES // GRID


def _fin_body(x_ref, w_ref, coef_ref, out_ref, col_v):
    i = pl.program_id(0)

    @pl.when(i == 0)
    def _():
        row = coef_ref[0:1, :] + coef_ref[8:9, :]
        denom = jnp.sum(row[:, :N_NODES])
        col_v[...] = jnp.transpose(row, (1, 0)) / denom

    wh = jnp.dot(x_ref[...], w_ref[...], preferred_element_type=jnp.float32)
    out_ref[...] = wh * col_v[pl.ds(i * BLKC, BLKC), :]


def kernel(x, edge_index, W, a):
    n, f = x.shape
    a12r = a[0, :, 0].reshape(2, f)

    s1t, s2t, m_t = pl.pallas_call(
        _mm_body,
        grid=(GRID,),
        in_specs=[
            pl.BlockSpec((BLK, f), lambda i: (i, 0)),
            pl.BlockSpec((f, f), lambda i: (0, 0)),
            pl.BlockSpec((2, f), lambda i: (0, 0)),
        ],
        out_specs=[
            pl.BlockSpec((8, BLK), lambda i: (0, i)),
            pl.BlockSpec((8, BLK), lambda i: (0, i)),
            pl.BlockSpec((8, 128), lambda i: (0, 0)),
        ],
        out_shape=[
            jax.ShapeDtypeStruct((8, N_PAD), jnp.float32),
            jax.ShapeDtypeStruct((8, N_PAD), jnp.float32),
            jax.ShapeDtypeStruct((8, 128), jnp.float32),
        ],
    )(x, W[0], a12r)

    ej = jnp.transpose(
        edge_index.astype(jnp.int32).reshape(2, E_ROWS, CHUNK), (1, 0, 2))

    coef_t = _edge_kernel()(s1t, s2t, m_t, ej)

    return pl.pallas_call(
        _fin_body,
        grid=(GRID,),
        in_specs=[
            pl.BlockSpec((BLKC, f), lambda i: (i, 0)),
            pl.BlockSpec((f, f), lambda i: (0, 0)),
            pl.BlockSpec((2 * 8, N_PAD), lambda i: (0, 0)),
        ],
        out_specs=pl.BlockSpec((BLKC, f), lambda i: (i, 0)),
        out_shape=jax.ShapeDtypeStruct((n, f), jnp.float32),
        scratch_shapes=[pltpu.VMEM((N_PAD, 1), jnp.float32)],
    )(x, W[0], coef_t)


# speedup vs baseline: 57.2531x; 1.0681x over previous
import functools

import jax
import jax.numpy as jnp
import numpy as np
from jax import lax
from jax.experimental import pallas as pl
from jax.experimental.pallas import tpu as pltpu
from jax.experimental.pallas import tpu_sc as plsc

ALPHA = 0.2
NEG_BIG = -3.0e38

NC = 2
NS = 16
NW = NC * NS
LANES = 16

N_NODES = 10000
N_PAD = 10240
ROWS = 80
CHUNK = 128
E_TILE_PAD = ROWS * CHUNK
E_PAD_TOTAL = NW * E_TILE_PAD
TBL_CHUNK = N_PAD // NS
SLICE = N_PAD // NS
GRID = 10
BLK = N_PAD // GRID

E_ROWS = 320000 // CHUNK
FULL_TILES = E_ROWS // ROWS
TAIL_ROWS = E_ROWS - FULL_TILES * ROWS


def _mm_body(x_ref, w_ref, a_ref, s1_ref, s2_ref, m_ref):
    i = pl.program_id(0)
    wh = jnp.dot(x_ref[...], w_ref[...], preferred_element_type=jnp.float32)
    srow = lax.dot_general(a_ref[...], wh, (((1,), (1,)), ((), ())),
                           preferred_element_type=jnp.float32)
    g = i * BLK + lax.broadcasted_iota(jnp.int32, (1, BLK), 1)
    valid = g < N_NODES
    srow_z = jnp.where(valid, srow, 0.0)
    z7 = jnp.zeros((7, BLK), jnp.float32)
    s1_ref[...] = jnp.concatenate([srow_z[0:1, :], z7], axis=0)
    s2_ref[...] = jnp.concatenate([srow_z[1:2, :], z7], axis=0)

    srow_m = jnp.where(valid, srow, NEG_BIG)
    ms1 = jnp.max(srow_m[0:1, :])
    ms2 = jnp.max(srow_m[1:2, :])
    mblk = jnp.concatenate(
        [jnp.full((4, 128), ms1), jnp.full((4, 128), ms2)], axis=0)

    @pl.when(i == 0)
    def _():
        m_ref[...] = jnp.full((8, 128), NEG_BIG)

    m_ref[...] = jnp.maximum(m_ref[...], mblk)


def _edge_body(s1_hbm, s2_hbm, m_hbm, ej_hbm,
               coef_out,
               s1_v, s2_v, m_v, ej_v, p_v,
               slice_v,
               coef_sh,
               sem):
    c = lax.axis_index("c")
    s = lax.axis_index("s")
    w = c * NS + s
    base = pl.multiple_of(w * ROWS, 8)

    for k in range(NS):
        r = s + k
        r = jnp.where(r >= NS, r - NS, r)
        off = pl.multiple_of(r * TBL_CHUNK, 8)
        pltpu.async_copy(s1_hbm.at[0, pl.ds(off, TBL_CHUNK)],
                         s1_v.at[pl.ds(off, TBL_CHUNK)], sem)
        pltpu.async_copy(s2_hbm.at[0, pl.ds(off, TBL_CHUNK)],
                         s2_v.at[pl.ds(off, TBL_CHUNK)], sem)

    @pl.when(w < FULL_TILES)
    def _():
        pltpu.async_copy(ej_hbm.at[pl.ds(base, ROWS)], ej_v, sem)

    @pl.when(w == FULL_TILES)
    def _():
        pltpu.async_copy(ej_hbm.at[pl.ds(base, TAIL_ROWS)],
                         ej_v.at[pl.ds(0, TAIL_ROWS)], sem)

    pltpu.sync_copy(m_hbm, m_v)

    z = jnp.zeros((LANES,), jnp.float32)
    for k in range(SLICE // LANES):
        slice_v[pl.ds(k * LANES, LANES)] = z

    @pl.when(w == FULL_TILES)
    def _():
        zi = jnp.zeros((LANES,), jnp.int32)
        lane = lax.iota(jnp.int32, LANES)

        def fill(j, carry):
            for k in range(CHUNK // LANES):
                dump = N_NODES + lax.rem(j * CHUNK + k * LANES + lane,
                                         N_PAD - N_NODES)
                ej_v[j, 0, pl.ds(k * LANES, LANES)] = zi
                ej_v[j, 1, pl.ds(k * LANES, LANES)] = dump
            return carry

        lax.fori_loop(TAIL_ROWS, ROWS, fill, 0)

    ms1 = jnp.max(m_v[0, pl.ds(0, LANES)])
    ms2 = jnp.max(m_v[4, pl.ds(0, LANES)])
    m = ms1 + ms2
    m = jnp.maximum(m, ALPHA * m)

    pltpu.make_async_copy(s1_hbm.at[0], s1_v, sem).wait()
    pltpu.make_async_copy(s2_hbm.at[0], s2_v, sem).wait()

    @pl.when(w < FULL_TILES)
    def _():
        pltpu.make_async_copy(ej_hbm.at[pl.ds(base, ROWS)], ej_v, sem).wait()

    @pl.when(w == FULL_TILES)
    def _():
        pltpu.make_async_copy(ej_hbm.at[pl.ds(base, TAIL_ROWS)],
                              ej_v.at[pl.ds(0, TAIL_ROWS)], sem).wait()

    pltpu.sync_copy(slice_v, coef_sh.at[pl.ds(s * SLICE, SLICE)])
    plsc.subcore_barrier()

    def edges(j, carry):
        for k in range(CHUNK // LANES):
            isrc = ej_v[j, 0, pl.ds(k * LANES, LANES)]
            idst = ej_v[j, 1, pl.ds(k * LANES, LANES)]
            ve = (plsc.load_gather(s1_v, [isrc])
                  + plsc.load_gather(s2_v, [idst]))
            ve = jnp.maximum(ve, ALPHA * ve)
            p_v[j, pl.ds(k * LANES, LANES)] = jnp.exp(ve - m)
        pltpu.async_copy(p_v.at[j], coef_sh.at[ej_v.at[j, 1]], sem, add=True)
        return carry

    lax.fori_loop(0, ROWS, edges, 0)
    pltpu.make_async_copy(s1_hbm.at[0], s1_v, sem).wait()
    plsc.subcore_barrier()

    pltpu.sync_copy(coef_sh.at[pl.ds(s * SLICE, SLICE)], slice_v)
    pltpu.sync_copy(
        slice_v,
        coef_out.at[pl.multiple_of(8 * c, 8), pl.ds(s * SLICE, SLICE)])


@functools.cache
def _edge_kernel():
    return functools.partial(
        pl.kernel,
        out_type=jax.ShapeDtypeStruct((2 * 8, N_PAD), jnp.float32),
        mesh=plsc.VectorSubcoreMesh(
            core_axis_name="c", subcore_axis_name="s", num_cores=NC, num_subcores=NS
        ),
        scratch_types=_edge_scratch(),
        compiler_params=pltpu.CompilerParams(needs_layout_passes=False),
    )(_edge_body)


def _edge_scratch():
    return [
        pltpu.VMEM((N_PAD,), jnp.float32),
        pltpu.VMEM((N_PAD,), jnp.float32),
        pltpu.VMEM((8, 128), jnp.float32),
        pltpu.VMEM((ROWS, 2, CHUNK), jnp.int32),
        pltpu.VMEM((ROWS, CHUNK), jnp.float32),
        pltpu.VMEM((SLICE,), jnp.float32),
        pltpu.VMEM_SHARED((N_PAD,), jnp.float32),
        pltpu.SemaphoreType.DMA,
    ]


def _fin_body(x_ref, w_ref, coef_ref, out_ref):
    wh = jnp.dot(x_ref[...], w_ref[...], preferred_element_type=jnp.float32)
    row = coef_ref[0:1, :] + coef_ref[8:9, :]
    denom = jnp.sum(row[:, :N_NODES])
    col = jnp.transpose(row, (1, 0))
    out_ref[...] = wh * (col[0:N_NODES, :] / denom)


def kernel(x, edge_index, W, a):
    n, f = x.shape
    a12r = a[0, :, 0].reshape(2, f)

    s1t, s2t, m_t = pl.pallas_call(
        _mm_body,
        grid=(GRID,),
        in_specs=[
            pl.BlockSpec((BLK, f), lambda i: (i, 0)),
            pl.BlockSpec((f, f), lambda i: (0, 0)),
            pl.BlockSpec((2, f), lambda i: (0, 0)),
        ],
        out_specs=[
            pl.BlockSpec((8, BLK), lambda i: (0, i)),
            pl.BlockSpec((8, BLK), lambda i: (0, i)),
            pl.BlockSpec((8, 128), lambda i: (0, 0)),
        ],
        out_shape=[
            jax.ShapeDtypeStruct((8, N_PAD), jnp.float32),
            jax.ShapeDtypeStruct((8, N_PAD), jnp.float32),
            jax.ShapeDtypeStruct((8, 128), jnp.float32),
        ],
    )(x, W[0], a12r)

    ej = jnp.transpose(
        edge_index.astype(jnp.int32).reshape(2, E_ROWS, CHUNK), (1, 0, 2))

    coef_t = _edge_kernel()(s1t, s2t, m_t, ej)

    return pl.pallas_call(
        _fin_body,
        out_shape=jax.ShapeDtypeStruct((n, f), jnp.float32),
    )(x, W[0], coef_t)
